# baseline (speedup 1.0000x reference)
import functools

import jax
import jax.numpy as jnp
from jax import lax
from jax.experimental import pallas as pl
from jax.experimental.pallas import tpu as pltpu
from jax.experimental.pallas import tpu_sc as plsc

N = 10000
E = 320000
D = 128
BN_EPS = 1e-5
BN_C = 1.0 / (1.0 + BN_EPS) ** 0.5

NC = 2
NS = 16
NW = NC * NS
CHUNK = 128
CPW = 80
E_PAD = NW * CPW * CHUNK
ROWS_PT = 632
N_SH = NS * ROWS_PT

_mesh = plsc.VectorSubcoreMesh(core_axis_name="c", subcore_axis_name="s")


@functools.partial(
    pl.kernel,
    out_type=jax.ShapeDtypeStruct((NW * N_SH,), jnp.float32),
    mesh=_mesh,
    scratch_types=[
        pltpu.VMEM((CPW, CHUNK), jnp.int32),
        pltpu.VMEM((N_SH,), jnp.float32),
    ],
    compiler_params=pltpu.CompilerParams(needs_layout_passes=False),
)
def _sc_degree(dst_hbm, zeros_hbm, out_hbm, dst_v, hist_v):
    c = lax.axis_index("c")
    s = lax.axis_index("s")
    w = s * NC + c
    pltpu.sync_copy(dst_hbm.at[pl.ds(w * CPW, CPW)], dst_v)
    pltpu.sync_copy(zeros_hbm, hist_v)
    ones = jnp.ones((16,), jnp.float32)

    def body(j, carry):
        for k in range(CHUNK // 16):
            idx = dst_v[j, pl.ds(k * 16, 16)]
            plsc.addupdate_scatter(hist_v, [idx], ones)
        return carry

    lax.fori_loop(0, CPW, body, 0)
    pltpu.sync_copy(hist_v, out_hbm.at[pl.ds(w * N_SH, N_SH)])


@functools.partial(
    pl.kernel,
    out_type=jax.ShapeDtypeStruct((NC, N_SH, D), jnp.float32),
    mesh=_mesh,
    scratch_types=[
        pltpu.VMEM((CPW // 2, CHUNK), jnp.int32),
        pltpu.VMEM((CPW // 2, CHUNK), jnp.int32),
        pltpu.VMEM((CHUNK, D), jnp.float32),
        pltpu.VMEM((CHUNK, D), jnp.float32),
        pltpu.VMEM_SHARED((N_SH, D), jnp.float32),
        pltpu.SemaphoreType.DMA,
        pltpu.SemaphoreType.DMA,
    ],
)
def _sc_edge_agg(t0_hbm, t1_hbm, src_hbm, dst_hbm, out_hbm, src_v, dst_v,
                 rows0, rows1, agg_sh, sem0, sem1):
    c = lax.axis_index("c")
    s = lax.axis_index("s")
    w = s * NC + c
    cpp = CPW // 2
    rows_slice = pl.ds(s * ROWS_PT, ROWS_PT)

    def _run(table):
        pltpu.sync_copy(table.at[rows_slice], agg_sh.at[rows_slice])
        plsc.subcore_barrier()

        def _step(j, rows, sem):
            pltpu.make_async_copy(table.at[pl.ds(0, CHUNK)], rows,
                                  sem).wait()
            pltpu.sync_copy(rows, agg_sh.at[dst_v.at[j]], add=True)

            @pl.when(j + 2 < cpp)
            def _():
                pltpu.async_copy(table.at[src_v.at[j + 2]], rows, sem)

        def body(jj, carry):
            _step(2 * jj, rows0, sem0)
            _step(2 * jj + 1, rows1, sem1)
            return carry

        for ph in range(2):
            base = w * CPW + ph * cpp
            pltpu.sync_copy(src_hbm.at[pl.ds(base, cpp)], src_v)
            pltpu.sync_copy(dst_hbm.at[pl.ds(base, cpp)], dst_v)
            pltpu.async_copy(table.at[src_v.at[0]], rows0, sem0)
            pltpu.async_copy(table.at[src_v.at[1]], rows1, sem1)
            lax.fori_loop(0, cpp // 2, body, 0)
        plsc.subcore_barrier()
        pltpu.sync_copy(agg_sh.at[rows_slice], out_hbm.at[c, rows_slice])

    @pl.when(c == 0)
    def _():
        _run(t0_hbm)

    @pl.when(c == 1)
    def _():
        _run(t1_hbm)


_R = 2000


def _tc_prep_body(dp_ref, x_ref, w_ref, dinv_ref, pa_ref, pb_ref):
    deg = jnp.sum(dp_ref[...], axis=1, keepdims=True) + 1.0
    di = lax.rsqrt(deg)
    dinv_ref[...] = jnp.broadcast_to(di, (_R, D))
    p = jnp.dot(x_ref[...], w_ref[...],
                preferred_element_type=jnp.float32) * di
    pa_ref[...] = p
    pb_ref[...] = p


def _tc_prep(degpart, x, W0):
    return pl.pallas_call(
        _tc_prep_body,
        grid=(N // _R,),
        in_specs=[
            pl.BlockSpec((_R, NW), lambda i: (i, 0)),
            pl.BlockSpec((_R, D), lambda i: (i, 0)),
            pl.BlockSpec((D, D), lambda i: (0, 0)),
        ],
        out_specs=[
            pl.BlockSpec((_R, D), lambda i: (i, 0)),
            pl.BlockSpec((_R, D), lambda i: (i, 0)),
            pl.BlockSpec((_R, D), lambda i: (i, 0)),
        ],
        out_shape=[
            jax.ShapeDtypeStruct((N, D), jnp.float32),
            jax.ShapeDtypeStruct((N, D), jnp.float32),
            jax.ShapeDtypeStruct((N, D), jnp.float32),
        ],
    )(degpart, x, W0)


def _tc_combine_body(pa_ref, p_ref, dinv_ref, res_ref, w_ref, b_ref, g_ref,
                     be_ref, h_ref, pna_ref, pnb_ref):
    agg = pa_ref[0] + pa_ref[1] - p_ref[...]
    gcn = agg * dinv_ref[...] + b_ref[...]
    h = jax.nn.relu(gcn * (BN_C * g_ref[...]) + be_ref[...]) + res_ref[...]
    h_ref[...] = h
    pn = jnp.dot(h, w_ref[...],
                 preferred_element_type=jnp.float32) * dinv_ref[...]
    pna_ref[...] = pn
    pnb_ref[...] = pn


def _tc_combine(part, p, dinvb, res, Wn, b, g, be):
    return pl.pallas_call(
        _tc_combine_body,
        grid=(N // _R,),
        in_specs=[
            pl.BlockSpec((NC, _R, D), lambda i: (0, i, 0)),
            pl.BlockSpec((_R, D), lambda i: (i, 0)),
            pl.BlockSpec((_R, D), lambda i: (i, 0)),
            pl.BlockSpec((_R, D), lambda i: (i, 0)),
            pl.BlockSpec((D, D), lambda i: (0, 0)),
            pl.BlockSpec((1, D), lambda i: (0, 0)),
            pl.BlockSpec((1, D), lambda i: (0, 0)),
            pl.BlockSpec((1, D), lambda i: (0, 0)),
        ],
        out_specs=[
            pl.BlockSpec((_R, D), lambda i: (i, 0)),
            pl.BlockSpec((_R, D), lambda i: (i, 0)),
            pl.BlockSpec((_R, D), lambda i: (i, 0)),
        ],
        out_shape=[
            jax.ShapeDtypeStruct((N, D), jnp.float32),
            jax.ShapeDtypeStruct((N, D), jnp.float32),
            jax.ShapeDtypeStruct((N, D), jnp.float32),
        ],
    )(part, p, dinvb, res, Wn, b.reshape(1, D), g.reshape(1, D),
      be.reshape(1, D))


def _tc_final_body(pa_ref, p_ref, dinv_ref, b_ref, out_ref):
    agg = pa_ref[0] + pa_ref[1] - p_ref[...]
    out_ref[...] = agg * dinv_ref[...] + b_ref[...]


def _tc_final(part, p, dinvb, b):
    return pl.pallas_call(
        _tc_final_body,
        grid=(N // _R,),
        in_specs=[
            pl.BlockSpec((NC, _R, D), lambda i: (0, i, 0)),
            pl.BlockSpec((_R, D), lambda i: (i, 0)),
            pl.BlockSpec((_R, D), lambda i: (i, 0)),
            pl.BlockSpec((1, D), lambda i: (0, 0)),
        ],
        out_specs=pl.BlockSpec((_R, D), lambda i: (i, 0)),
        out_shape=jax.ShapeDtypeStruct((N, D), jnp.float32),
    )(part, p, dinvb, b.reshape(1, D))


def _pad_table(p):
    return jnp.concatenate([p, jnp.zeros((N_SH - N, D), jnp.float32)],
                           axis=0)


@jax.jit
def kernel(x, edge_index, W0, b0, W1, b1, W2, b2, g0, be0, g1, be1):
    ei = edge_index.astype(jnp.int32)
    pad = jnp.full((E_PAD - E,), N, jnp.int32)
    src2d = jnp.concatenate([ei[0], pad]).reshape(NW * CPW, CHUNK)
    dst2d = jnp.concatenate([ei[1], pad]).reshape(NW * CPW, CHUNK)
    zeros_hbm = jnp.zeros((N_SH,), jnp.float32)

    degpart = _sc_degree(dst2d, zeros_hbm)
    dp_t = degpart.reshape(NW, N_SH).T
    dinvb, p0a, p0b = _tc_prep(dp_t, x, W0)

    part0 = _sc_edge_agg(_pad_table(p0a), _pad_table(p0b), src2d, dst2d)
    h1, p1a, p1b = _tc_combine(part0, p0a, dinvb, x, W1, b0, g0, be0)

    part1 = _sc_edge_agg(_pad_table(p1a), _pad_table(p1b), src2d, dst2d)
    h2, p2a, p2b = _tc_combine(part1, p1a, dinvb, h1, W2, b1, g1, be1)

    part2 = _sc_edge_agg(_pad_table(p2a), _pad_table(p2b), src2d, dst2d)
    return _tc_final(part2, p2a, dinvb, b2)


# speedup vs baseline: 1.4070x; 1.4070x over previous
import functools

import jax
import jax.numpy as jnp
from jax import lax
from jax.experimental import pallas as pl
from jax.experimental.pallas import tpu as pltpu
from jax.experimental.pallas import tpu_sc as plsc

N = 10000
E = 320000
D = 128
BN_EPS = 1e-5
BN_C = 1.0 / (1.0 + BN_EPS) ** 0.5

NC = 2
NS = 16
NW = NC * NS
CHUNK = 128
CPW = 80
E_PAD = NW * CPW * CHUNK
ROWS_PT = 632
N_SH = NS * ROWS_PT
DH = D // 2

_mesh = plsc.VectorSubcoreMesh(core_axis_name="c", subcore_axis_name="s")


@functools.partial(
    pl.kernel,
    out_type=jax.ShapeDtypeStruct((NW * N_SH,), jnp.float32),
    mesh=_mesh,
    scratch_types=[
        pltpu.VMEM((CPW, CHUNK), jnp.int32),
        pltpu.VMEM((N_SH,), jnp.float32),
    ],
    compiler_params=pltpu.CompilerParams(needs_layout_passes=False),
)
def _sc_degree(dst_hbm, zeros_hbm, out_hbm, dst_v, hist_v):
    c = lax.axis_index("c")
    s = lax.axis_index("s")
    w = s * NC + c
    pltpu.sync_copy(dst_hbm.at[pl.ds(w * CPW, CPW)], dst_v)
    pltpu.sync_copy(zeros_hbm, hist_v)
    ones = jnp.ones((16,), jnp.float32)

    def body(j, carry):
        for k in range(CHUNK // 16):
            idx = dst_v[j, pl.ds(k * 16, 16)]
            plsc.addupdate_scatter(hist_v, [idx], ones)
        return carry

    lax.fori_loop(0, CPW, body, 0)
    pltpu.sync_copy(hist_v, out_hbm.at[pl.ds(w * N_SH, N_SH)])


_TPW = 2 * CPW
_NBUF = 4


@functools.partial(
    pl.kernel,
    out_type=jax.ShapeDtypeStruct((NC, N_SH, DH), jnp.float32),
    mesh=_mesh,
    scratch_types=[
        pltpu.VMEM((_TPW, CHUNK), jnp.int32),
        pltpu.VMEM((_TPW, CHUNK), jnp.int32),
        [pltpu.VMEM((CHUNK, DH), jnp.float32)] * _NBUF,
        pltpu.VMEM_SHARED((N_SH, DH), jnp.float32),
        [pltpu.SemaphoreType.DMA] * _NBUF,
    ],
    compiler_params=pltpu.CompilerParams(use_tc_tiling_on_sc=False),
)
def _sc_edge_agg(t_hbm, src_hbm, dst_hbm, out_hbm, src_v, dst_v,
                 rows, agg_sh, sems):
    c = lax.axis_index("c")
    s = lax.axis_index("s")
    table = t_hbm.at[c]
    pltpu.sync_copy(src_hbm.at[pl.ds(s * _TPW, _TPW)], src_v)
    pltpu.sync_copy(dst_hbm.at[pl.ds(s * _TPW, _TPW)], dst_v)
    rows_slice = pl.ds(s * ROWS_PT, ROWS_PT)
    pltpu.sync_copy(table.at[rows_slice], agg_sh.at[rows_slice])
    plsc.subcore_barrier()

    def _step(j, k):
        pltpu.make_async_copy(table.at[pl.ds(0, CHUNK)], rows[k],
                              sems[k]).wait()
        pltpu.sync_copy(rows[k], agg_sh.at[dst_v.at[j]], add=True)

        @pl.when(j + _NBUF < _TPW)
        def _():
            pltpu.async_copy(table.at[src_v.at[j + _NBUF]], rows[k], sems[k])

    def body(jj, carry):
        for k in range(_NBUF):
            _step(_NBUF * jj + k, k)
        return carry

    for k in range(_NBUF):
        pltpu.async_copy(table.at[src_v.at[k]], rows[k], sems[k])
    lax.fori_loop(0, _TPW // _NBUF, body, 0)
    plsc.subcore_barrier()
    pltpu.sync_copy(agg_sh.at[rows_slice], out_hbm.at[c, rows_slice])


_R = N_SH // 8


def _row_mask(i):
    rows = lax.broadcasted_iota(jnp.int32, (_R, 1), 0) + i * _R
    return rows < N


def _split_table(p, t_ref):
    t_ref[0] = p[:, :DH]
    t_ref[1] = p[:, DH:]


def _tc_prep_body(dp_ref, x_ref, w_ref, dinv_ref, t_ref):
    deg = jnp.sum(dp_ref[...], axis=1, keepdims=True) + 1.0
    di = lax.rsqrt(deg)
    dinv_ref[...] = jnp.broadcast_to(di, (_R, D))
    p = jnp.dot(x_ref[...], w_ref[...],
                preferred_element_type=jnp.float32) * di
    _split_table(p, t_ref)


def _tc_prep(degpart, xp, W0):
    return pl.pallas_call(
        _tc_prep_body,
        grid=(N_SH // _R,),
        in_specs=[
            pl.BlockSpec((_R, NW), lambda i: (i, 0)),
            pl.BlockSpec((_R, D), lambda i: (i, 0)),
            pl.BlockSpec((D, D), lambda i: (0, 0)),
        ],
        out_specs=[
            pl.BlockSpec((_R, D), lambda i: (i, 0)),
            pl.BlockSpec((NC, _R, DH), lambda i: (0, i, 0)),
        ],
        out_shape=[
            jax.ShapeDtypeStruct((N_SH, D), jnp.float32),
            jax.ShapeDtypeStruct((NC, N_SH, DH), jnp.float32),
        ],
    )(degpart, xp, W0)


def _agg_from_parts(pa):
    return jnp.concatenate([pa[0], pa[1]], axis=-1)


def _tc_combine_body(pa_ref, dinv_ref, res_ref, w_ref, b_ref, g_ref,
                     be_ref, h_ref, t_ref):
    i = pl.program_id(0)
    agg = _agg_from_parts(pa_ref[...])
    gcn = agg * dinv_ref[...] + b_ref[...]
    h = jax.nn.relu(gcn * (BN_C * g_ref[...]) + be_ref[...]) + res_ref[...]
    h = jnp.where(_row_mask(i), h, 0.0)
    h_ref[...] = h
    pn = jnp.dot(h, w_ref[...],
                 preferred_element_type=jnp.float32) * dinv_ref[...]
    _split_table(pn, t_ref)


def _tc_combine(part, dinvb, res, Wn, b, g, be):
    return pl.pallas_call(
        _tc_combine_body,
        grid=(N_SH // _R,),
        in_specs=[
            pl.BlockSpec((NC, _R, DH), lambda i: (0, i, 0)),
            pl.BlockSpec((_R, D), lambda i: (i, 0)),
            pl.BlockSpec((_R, D), lambda i: (i, 0)),
            pl.BlockSpec((D, D), lambda i: (0, 0)),
            pl.BlockSpec((1, D), lambda i: (0, 0)),
            pl.BlockSpec((1, D), lambda i: (0, 0)),
            pl.BlockSpec((1, D), lambda i: (0, 0)),
        ],
        out_specs=[
            pl.BlockSpec((_R, D), lambda i: (i, 0)),
            pl.BlockSpec((NC, _R, DH), lambda i: (0, i, 0)),
        ],
        out_shape=[
            jax.ShapeDtypeStruct((N_SH, D), jnp.float32),
            jax.ShapeDtypeStruct((NC, N_SH, DH), jnp.float32),
        ],
    )(part, dinvb, res, Wn, b.reshape(1, D), g.reshape(1, D),
      be.reshape(1, D))


_RF = 2000


def _tc_final_body(pa_ref, dinv_ref, b_ref, out_ref):
    agg = _agg_from_parts(pa_ref[...])
    out_ref[...] = agg * dinv_ref[...] + b_ref[...]


def _tc_final(part, dinvb, b):
    return pl.pallas_call(
        _tc_final_body,
        grid=(N // _RF,),
        in_specs=[
            pl.BlockSpec((NC, _RF, DH), lambda i: (0, i, 0)),
            pl.BlockSpec((_RF, D), lambda i: (i, 0)),
            pl.BlockSpec((1, D), lambda i: (0, 0)),
        ],
        out_specs=pl.BlockSpec((_RF, D), lambda i: (i, 0)),
        out_shape=jax.ShapeDtypeStruct((N, D), jnp.float32),
    )(part, dinvb, b.reshape(1, D))


@jax.jit
def kernel(x, edge_index, W0, b0, W1, b1, W2, b2, g0, be0, g1, be1):
    ei = edge_index.astype(jnp.int32)
    pad = jnp.full((E_PAD - E,), N, jnp.int32)
    src2d = jnp.concatenate([ei[0], pad]).reshape(NW * CPW, CHUNK)
    dst2d = jnp.concatenate([ei[1], pad]).reshape(NW * CPW, CHUNK)
    zeros_hbm = jnp.zeros((N_SH,), jnp.float32)
    xp = jnp.concatenate([x, jnp.zeros((N_SH - N, D), jnp.float32)], axis=0)

    degpart = _sc_degree(dst2d, zeros_hbm)
    dp_t = degpart.reshape(NW, N_SH).T
    dinvb, t0 = _tc_prep(dp_t, xp, W0)

    part0 = _sc_edge_agg(t0, src2d, dst2d)
    h1, t1 = _tc_combine(part0, dinvb, xp, W1, b0, g0, be0)

    part1 = _sc_edge_agg(t1, src2d, dst2d)
    h2, t2 = _tc_combine(part1, dinvb, h1, W2, b1, g1, be1)

    part2 = _sc_edge_agg(t2, src2d, dst2d)
    return _tc_final(part2, dinvb, b2)
